# Initial kernel scaffold; baseline (speedup 1.0000x reference)
import jax
import jax.numpy as jnp
from jax.experimental import pallas as pl


def kernel(exercises, categories, exercise_embed, category_embed, position_embed):
    raise NotImplementedError("write your pallas kernel here")


# speedup vs baseline: 4.3030x; 4.3030x over previous
import functools

import jax
import jax.numpy as jnp
from jax import lax
from jax.experimental import pallas as pl
from jax.experimental.pallas import tpu as pltpu
from jax.experimental.pallas import tpu_sc as plsc

N_EX = 100000
N_CAT = 1000
D = 64
SEQ = 200
B = 4096

ROWS = B * SEQ
CHUNK = 128
NW = 32
NCHUNKS = ROWS // CHUNK
CPW = NCHUNKS // NW
LANES = 16


@functools.partial(
    pl.kernel,
    mesh=plsc.VectorSubcoreMesh(core_axis_name="c", subcore_axis_name="s"),
    out_type=jax.ShapeDtypeStruct((ROWS, D), jnp.float32),
    compiler_params=pltpu.CompilerParams(use_tc_tiling_on_sc=False),
    scratch_types=[
        pltpu.VMEM((CPW, CHUNK), jnp.int32),
        pltpu.VMEM((CPW, CHUNK), jnp.int32),
        pltpu.VMEM((SEQ, D), jnp.float32),
        pltpu.VMEM((CHUNK, D), jnp.float32),
        pltpu.VMEM((CHUNK, D), jnp.float32),
        pltpu.SemaphoreType.DMA,
        pltpu.SemaphoreType.DMA,
    ],
)
def _emb_kernel(ex_idx_hbm, cat_idx_hbm, ex_tab, cat_tab, pos_hbm, out_hbm,
                eidx, cidx, pos_v, exb, catb, sem_e, sem_c):
    wid = lax.axis_index("s") * 2 + lax.axis_index("c")
    base = wid * CPW

    pltpu.sync_copy(ex_idx_hbm.at[pl.ds(base, CPW)], eidx)
    pltpu.sync_copy(cat_idx_hbm.at[pl.ds(base, CPW)], cidx)
    pltpu.sync_copy(pos_hbm, pos_v)

    def chunk_body(j, carry):
        ce = pltpu.async_copy(ex_tab.at[eidx.at[j]], exb, sem_e)
        cc = pltpu.async_copy(cat_tab.at[cidx.at[j]], catb, sem_c)
        ce.wait()
        cc.wait()
        p0 = lax.rem((base + j) * CHUNK, SEQ)

        def row_body(i, c2):
            p = p0 + i
            p = jnp.where(p >= SEQ, p - SEQ, p)
            for c in range(D // LANES):
                s = pl.ds(c * LANES, LANES)
                exb[i, s] = exb[i, s] + catb[i, s] + pos_v[p, s]
            return c2

        lax.fori_loop(0, CHUNK, row_body, 0)
        pltpu.sync_copy(exb, out_hbm.at[pl.ds((base + j) * CHUNK, CHUNK)])
        return carry

    lax.fori_loop(0, CPW, chunk_body, 0)


def kernel(exercises, categories, exercise_embed, category_embed,
           position_embed):
    ex_idx = exercises.reshape(NCHUNKS, CHUNK).astype(jnp.int32)
    cat_idx = categories.reshape(NCHUNKS, CHUNK).astype(jnp.int32)
    out = _emb_kernel(ex_idx, cat_idx, exercise_embed, category_embed,
                      position_embed)
    return out.reshape(B, SEQ, D)


# speedup vs baseline: 5.5598x; 1.2921x over previous
import functools

import jax
import jax.numpy as jnp
from jax import lax
from jax.experimental import pallas as pl
from jax.experimental.pallas import tpu as pltpu
from jax.experimental.pallas import tpu_sc as plsc

N_EX = 100000
N_CAT = 1000
D = 64
SEQ = 200
B = 4096

ROWS = B * SEQ
CHUNK = 128
NW = 32
NCHUNKS = ROWS // CHUNK
CPW = NCHUNKS // NW
LANES = 16
HALF = CPW // 2


@functools.partial(
    pl.kernel,
    mesh=plsc.VectorSubcoreMesh(core_axis_name="c", subcore_axis_name="s"),
    out_type=jax.ShapeDtypeStruct((ROWS, D), jnp.float32),
    compiler_params=pltpu.CompilerParams(use_tc_tiling_on_sc=False),
    scratch_types=[
        pltpu.VMEM((CPW, CHUNK), jnp.int32),
        pltpu.VMEM((CPW, CHUNK), jnp.int32),
        pltpu.VMEM((SEQ, D), jnp.float32),
        pltpu.VMEM((CHUNK, D), jnp.float32),
        pltpu.VMEM((CHUNK, D), jnp.float32),
        pltpu.VMEM((CHUNK, D), jnp.float32),
        pltpu.VMEM((CHUNK, D), jnp.float32),
        pltpu.VMEM((CHUNK, D), jnp.float32),
        pltpu.VMEM((CHUNK, D), jnp.float32),
        pltpu.SemaphoreType.DMA,
        pltpu.SemaphoreType.DMA,
        pltpu.SemaphoreType.DMA,
        pltpu.SemaphoreType.DMA,
        pltpu.SemaphoreType.DMA,
        pltpu.SemaphoreType.DMA,
    ],
)
def _emb_kernel(ex_idx_hbm, cat_idx_hbm, ex_tab, cat_tab, pos_hbm, out_hbm,
                eidx, cidx, pos_v, exb0, catb0, res0, exb1, catb1, res1,
                sem_e0, sem_c0, sem_o0, sem_e1, sem_c1, sem_o1):
    wid = lax.axis_index("s") * 2 + lax.axis_index("c")
    base = wid * CPW

    pltpu.sync_copy(ex_idx_hbm.at[pl.ds(base, CPW)], eidx)
    pltpu.sync_copy(cat_idx_hbm.at[pl.ds(base, CPW)], cidx)
    pltpu.sync_copy(pos_hbm, pos_v)

    def gathers(j, exb, catb, sem_e, sem_c):
        pltpu.async_copy(ex_tab.at[eidx.at[j]], exb, sem_e)
        pltpu.async_copy(cat_tab.at[cidx.at[j]], catb, sem_c)

    def wait_gathers(j, exb, catb, sem_e, sem_c):
        pltpu.make_async_copy(ex_tab.at[eidx.at[j]], exb, sem_e).wait()
        pltpu.make_async_copy(cat_tab.at[cidx.at[j]], catb, sem_c).wait()

    def valu(j, exb, catb, res):
        p0 = lax.rem((base + j) * CHUNK, SEQ)

        def row_body(i, c2):
            p = p0 + i
            p = jnp.where(p >= SEQ, p - SEQ, p)
            for c in range(D // LANES):
                s = pl.ds(c * LANES, LANES)
                res[i, s] = exb[i, s] + catb[i, s] + pos_v[p, s]
            return c2

        lax.fori_loop(0, CHUNK, row_body, 0)

    def out_ref(j):
        return out_hbm.at[pl.ds((base + j) * CHUNK, CHUNK)]

    gathers(0, exb0, catb0, sem_e0, sem_c0)
    gathers(1, exb1, catb1, sem_e1, sem_c1)

    def loop_body(t, carry):
        a = 2 * t

        wait_gathers(a, exb0, catb0, sem_e0, sem_c0)

        @pl.when(t > 0)
        def _():
            pltpu.make_async_copy(res0, out_ref(a - 2), sem_o0).wait()

        valu(a, exb0, catb0, res0)
        pltpu.async_copy(res0, out_ref(a), sem_o0)

        @pl.when(t < HALF - 1)
        def _():
            gathers(a + 2, exb0, catb0, sem_e0, sem_c0)

        wait_gathers(a + 1, exb1, catb1, sem_e1, sem_c1)

        @pl.when(t > 0)
        def _():
            pltpu.make_async_copy(res1, out_ref(a - 1), sem_o1).wait()

        valu(a + 1, exb1, catb1, res1)
        pltpu.async_copy(res1, out_ref(a + 1), sem_o1)

        @pl.when(t < HALF - 1)
        def _():
            gathers(a + 3, exb1, catb1, sem_e1, sem_c1)

        return carry

    lax.fori_loop(0, HALF, loop_body, 0)

    pltpu.make_async_copy(res0, out_ref(CPW - 2), sem_o0).wait()
    pltpu.make_async_copy(res1, out_ref(CPW - 1), sem_o1).wait()


def kernel(exercises, categories, exercise_embed, category_embed,
           position_embed):
    ex_idx = exercises.reshape(NCHUNKS, CHUNK).astype(jnp.int32)
    cat_idx = categories.reshape(NCHUNKS, CHUNK).astype(jnp.int32)
    out = _emb_kernel(ex_idx, cat_idx, exercise_embed, category_embed,
                      position_embed)
    return out.reshape(B, SEQ, D)


# speedup vs baseline: 6.2530x; 1.1247x over previous
import functools

import jax
import jax.numpy as jnp
from jax import lax
from jax.experimental import pallas as pl
from jax.experimental.pallas import tpu as pltpu
from jax.experimental.pallas import tpu_sc as plsc

N_EX = 100000
N_CAT = 1000
D = 64
SEQ = 200
B = 4096

ROWS = B * SEQ
CHUNK = 100
NW = 32
NCHUNKS = ROWS // CHUNK
CPW = NCHUNKS // NW
BPW = B // NW
LANES = 16
HALF = CPW // 2


@functools.partial(
    pl.kernel,
    mesh=plsc.VectorSubcoreMesh(core_axis_name="c", subcore_axis_name="s"),
    out_type=jax.ShapeDtypeStruct((B, SEQ, D), jnp.float32),
    compiler_params=pltpu.CompilerParams(use_tc_tiling_on_sc=False),
    scratch_types=[
        pltpu.VMEM((CPW, CHUNK), jnp.int32),
        pltpu.VMEM((CPW, CHUNK), jnp.int32),
        pltpu.VMEM((SEQ, D), jnp.float32),
        pltpu.VMEM((CHUNK, D), jnp.float32),
        pltpu.VMEM((CHUNK, D), jnp.float32),
        pltpu.VMEM((CHUNK, D), jnp.float32),
        pltpu.VMEM((CHUNK, D), jnp.float32),
        pltpu.VMEM((CHUNK, D), jnp.float32),
        pltpu.VMEM((CHUNK, D), jnp.float32),
        pltpu.SemaphoreType.DMA,
        pltpu.SemaphoreType.DMA,
        pltpu.SemaphoreType.DMA,
        pltpu.SemaphoreType.DMA,
        pltpu.SemaphoreType.DMA,
        pltpu.SemaphoreType.DMA,
    ],
)
def _emb_kernel(ex_idx_hbm, cat_idx_hbm, ex_tab, cat_tab, pos_hbm, out_hbm,
                eidx, cidx, pos_v, exb0, catb0, res0, exb1, catb1, res1,
                sem_e0, sem_c0, sem_o0, sem_e1, sem_c1, sem_o1):
    wid = lax.axis_index("s") * 2 + lax.axis_index("c")
    base = wid * CPW
    bbase = wid * BPW

    pltpu.sync_copy(ex_idx_hbm.at[pl.ds(base, CPW)], eidx)
    pltpu.sync_copy(cat_idx_hbm.at[pl.ds(base, CPW)], cidx)
    pltpu.sync_copy(pos_hbm, pos_v)

    def gathers(j, exb, catb, sem_e, sem_c):
        pltpu.async_copy(ex_tab.at[eidx.at[j]], exb, sem_e)
        pltpu.async_copy(cat_tab.at[cidx.at[j]], catb, sem_c)

    def wait_gathers(j, exb, catb, sem_e, sem_c):
        pltpu.make_async_copy(ex_tab.at[eidx.at[j]], exb, sem_e).wait()
        pltpu.make_async_copy(cat_tab.at[cidx.at[j]], catb, sem_c).wait()

    def valu(exb, catb, res, pos_off):
        def row_body(i, c2):
            for c in range(D // LANES):
                s = pl.ds(c * LANES, LANES)
                res[i, s] = exb[i, s] + catb[i, s] + pos_v[pos_off + i, s]
            return c2

        lax.fori_loop(0, CHUNK, row_body, 0)

    def out_ref(t, half):
        return out_hbm.at[bbase + t, pl.ds(half * CHUNK, CHUNK)]

    gathers(0, exb0, catb0, sem_e0, sem_c0)
    gathers(1, exb1, catb1, sem_e1, sem_c1)

    def loop_body(t, carry):
        a = 2 * t

        wait_gathers(a, exb0, catb0, sem_e0, sem_c0)

        @pl.when(t > 0)
        def _():
            pltpu.make_async_copy(res0, out_ref(t - 1, 0), sem_o0).wait()

        valu(exb0, catb0, res0, 0)
        pltpu.async_copy(res0, out_ref(t, 0), sem_o0)

        @pl.when(t < HALF - 1)
        def _():
            gathers(a + 2, exb0, catb0, sem_e0, sem_c0)

        wait_gathers(a + 1, exb1, catb1, sem_e1, sem_c1)

        @pl.when(t > 0)
        def _():
            pltpu.make_async_copy(res1, out_ref(t - 1, 1), sem_o1).wait()

        valu(exb1, catb1, res1, CHUNK)
        pltpu.async_copy(res1, out_ref(t, 1), sem_o1)

        @pl.when(t < HALF - 1)
        def _():
            gathers(a + 3, exb1, catb1, sem_e1, sem_c1)

        return carry

    lax.fori_loop(0, HALF, loop_body, 0)

    pltpu.make_async_copy(res0, out_ref(HALF - 1, 0), sem_o0).wait()
    pltpu.make_async_copy(res1, out_ref(HALF - 1, 1), sem_o1).wait()


def kernel(exercises, categories, exercise_embed, category_embed,
           position_embed):
    ex_idx = exercises.reshape(NCHUNKS, CHUNK).astype(jnp.int32)
    cat_idx = categories.reshape(NCHUNKS, CHUNK).astype(jnp.int32)
    return _emb_kernel(ex_idx, cat_idx, exercise_embed, category_embed,
                       position_embed)
